# baseline (speedup 1.0000x reference)
import functools
import jax
import jax.numpy as jnp
from jax.experimental import pallas as pl

_NEG_SLOPE = 0.2
_BLK = 1000


def _proj_body(x_ref, wl_ref, bl_ref, wr_ref, br_ref, xl_ref, xr_ref):
    xb = x_ref[...]
    xl_ref[...] = jnp.dot(xb, wl_ref[...], preferred_element_type=jnp.float32) + bl_ref[...]
    xr_ref[...] = jnp.dot(xb, wr_ref[...], preferred_element_type=jnp.float32) + br_ref[...]


def _project(x, wlT, bl, wrT, br):
    n, f = x.shape
    h = wlT.shape[1]
    grid = (n // _BLK,)
    return pl.pallas_call(
        _proj_body,
        grid=grid,
        in_specs=[
            pl.BlockSpec((_BLK, f), lambda i: (i, 0)),
            pl.BlockSpec((f, h), lambda i: (0, 0)),
            pl.BlockSpec((1, h), lambda i: (0, 0)),
            pl.BlockSpec((f, h), lambda i: (0, 0)),
            pl.BlockSpec((1, h), lambda i: (0, 0)),
        ],
        out_specs=[
            pl.BlockSpec((_BLK, h), lambda i: (i, 0)),
            pl.BlockSpec((_BLK, h), lambda i: (i, 0)),
        ],
        out_shape=[
            jax.ShapeDtypeStruct((n, h), jnp.float32),
            jax.ShapeDtypeStruct((n, h), jnp.float32),
        ],
    )(x, wlT, bl.reshape(1, h), wrT, br.reshape(1, h))


def _edgeproj_body(ea_ref, we_ref, out_ref):
    out_ref[...] = jnp.dot(ea_ref[...], we_ref[...], preferred_element_type=jnp.float32)


def _edge_project(ea2, weT):
    e2, d = ea2.shape
    h = weT.shape[1]
    return pl.pallas_call(
        _edgeproj_body,
        grid=(e2 // _BLK,),
        in_specs=[
            pl.BlockSpec((_BLK, d), lambda i: (i, 0)),
            pl.BlockSpec((d, h), lambda i: (0, 0)),
        ],
        out_specs=pl.BlockSpec((_BLK, h), lambda i: (i, 0)),
        out_shape=jax.ShapeDtypeStruct((e2, h), jnp.float32),
    )(ea2, weT)


def _logit_body(xls_ref, xrd_ref, e_ref, att_ref, out_ref):
    m = xls_ref[...] + xrd_ref[...] + e_ref[...]
    g = jnp.where(m >= 0, m, _NEG_SLOPE * m)
    out_ref[...] = jnp.sum(g * att_ref[...], axis=1, keepdims=True)


def _edge_logits(xls, xrd, e2v, att):
    e2, h = xls.shape
    return pl.pallas_call(
        _logit_body,
        grid=(e2 // _BLK,),
        in_specs=[
            pl.BlockSpec((_BLK, h), lambda i: (i, 0)),
            pl.BlockSpec((_BLK, h), lambda i: (i, 0)),
            pl.BlockSpec((_BLK, h), lambda i: (i, 0)),
            pl.BlockSpec((1, h), lambda i: (0, 0)),
        ],
        out_specs=pl.BlockSpec((_BLK, 1), lambda i: (i, 0)),
        out_shape=jax.ShapeDtypeStruct((e2, 1), jnp.float32),
    )(xls, xrd, e2v, att.reshape(1, h))


def _z_body(l_ref, mx_ref, out_ref):
    out_ref[...] = jnp.exp(l_ref[...] - mx_ref[...])


def _edge_z(logit, mxd):
    e2 = logit.shape[0]
    return pl.pallas_call(
        _z_body,
        grid=(e2 // _BLK,),
        in_specs=[
            pl.BlockSpec((_BLK, 1), lambda i: (i, 0)),
            pl.BlockSpec((_BLK, 1), lambda i: (i, 0)),
        ],
        out_specs=pl.BlockSpec((_BLK, 1), lambda i: (i, 0)),
        out_shape=jax.ShapeDtypeStruct((e2, 1), jnp.float32),
    )(logit, mxd)


def _alpha_body(z_ref, den_ref, xls_ref, out_ref):
    out_ref[...] = (z_ref[...] / (den_ref[...] + 1e-16)) * xls_ref[...]


def _edge_weighted(z, dend, xls):
    e2, h = xls.shape
    return pl.pallas_call(
        _alpha_body,
        grid=(e2 // _BLK,),
        in_specs=[
            pl.BlockSpec((_BLK, 1), lambda i: (i, 0)),
            pl.BlockSpec((_BLK, 1), lambda i: (i, 0)),
            pl.BlockSpec((_BLK, h), lambda i: (i, 0)),
        ],
        out_specs=pl.BlockSpec((_BLK, h), lambda i: (i, 0)),
        out_shape=jax.ShapeDtypeStruct((e2, h), jnp.float32),
    )(z, dend, xls)


def _gatv2_layer(x, s2, d2, e2v, wl, bl, wr, br, att, b, n):
    xl, xr = _project(x, wl.T, bl, wr.T, br)
    xls = jnp.take(xl, s2, axis=0)
    xrd = jnp.take(xr, d2, axis=0)
    logit = _edge_logits(xls, xrd, e2v, att)[:, 0]
    mx = jax.ops.segment_max(logit, d2, num_segments=n)
    z = _edge_z(logit[:, None], jnp.take(mx, d2)[:, None])[:, 0]
    den = jax.ops.segment_sum(z, d2, num_segments=n)
    p = _edge_weighted(z[:, None], jnp.take(den, d2)[:, None], xls)
    out = jax.ops.segment_sum(p, d2, num_segments=n)
    return out + b


def kernel(x, edge_index, edge_attr,
           Wl1, bl1, Wr1, br1, We1, att1, b1,
           Wl2, bl2, Wr2, br2, We2, att2, b2,
           Wl3, bl3, Wr3, br3, We3, att3, b3):
    n = x.shape[0]
    src, dst = edge_index[0], edge_index[1]
    sums = jax.ops.segment_sum(edge_attr, dst, num_segments=n)
    cnt = jax.ops.segment_sum(jnp.ones((edge_attr.shape[0],), jnp.float32), dst,
                              num_segments=n)
    loop_ea = sums / jnp.maximum(cnt, 1.0)[:, None]
    loop_idx = jnp.arange(n, dtype=src.dtype)
    s2 = jnp.concatenate([src, loop_idx])
    d2 = jnp.concatenate([dst, loop_idx])
    ea2 = jnp.concatenate([edge_attr, loop_ea], axis=0)

    k1, k2 = jax.random.split(jax.random.key(42))

    h = _gatv2_layer(x, s2, d2, _edge_project(ea2, We1.T), Wl1, bl1, Wr1, br1,
                     att1, b1, n)
    h = jax.nn.relu(h)
    keep = jax.random.bernoulli(k1, 0.8, h.shape)
    h = jnp.where(keep, h / 0.8, jnp.zeros_like(h))

    h = _gatv2_layer(h, s2, d2, _edge_project(ea2, We2.T), Wl2, bl2, Wr2, br2,
                     att2, b2, n)
    h = jax.nn.relu(h)
    keep = jax.random.bernoulli(k2, 0.8, h.shape)
    h = jnp.where(keep, h / 0.8, jnp.zeros_like(h))

    h = _gatv2_layer(h, s2, d2, _edge_project(ea2, We3.T), Wl3, bl3, Wr3, br3,
                     att3, b3, n)
    return h


# speedup vs baseline: 1.0638x; 1.0638x over previous
import functools
import jax
import jax.numpy as jnp
from jax.experimental import pallas as pl

_NEG_SLOPE = 0.2
_BLK = 2000


def _proj_body(x_ref, wl_ref, bl_ref, wr_ref, br_ref, xl_ref, xr_ref):
    xb = x_ref[...]
    xl = jnp.dot(xb, wl_ref[...], preferred_element_type=jnp.float32) + bl_ref[...]
    xr = jnp.dot(xb, wr_ref[...], preferred_element_type=jnp.float32) + br_ref[...]
    xl_ref[...] = xl.astype(jnp.bfloat16)
    xr_ref[...] = xr.astype(jnp.bfloat16)


def _project(x, wlT, bl, wrT, br):
    n, f = x.shape
    h = wlT.shape[1]
    grid = (n // _BLK,)
    return pl.pallas_call(
        _proj_body,
        grid=grid,
        in_specs=[
            pl.BlockSpec((_BLK, f), lambda i: (i, 0)),
            pl.BlockSpec((f, h), lambda i: (0, 0)),
            pl.BlockSpec((1, h), lambda i: (0, 0)),
            pl.BlockSpec((f, h), lambda i: (0, 0)),
            pl.BlockSpec((1, h), lambda i: (0, 0)),
        ],
        out_specs=[
            pl.BlockSpec((_BLK, h), lambda i: (i, 0)),
            pl.BlockSpec((_BLK, h), lambda i: (i, 0)),
        ],
        out_shape=[
            jax.ShapeDtypeStruct((n, h), jnp.bfloat16),
            jax.ShapeDtypeStruct((n, h), jnp.bfloat16),
        ],
    )(x, wlT, bl.reshape(1, h), wrT, br.reshape(1, h))


def _edgeproj_body(ea_ref, we_ref, out_ref):
    out_ref[...] = jnp.dot(
        ea_ref[...], we_ref[...], preferred_element_type=jnp.float32
    ).astype(jnp.bfloat16)


def _edge_project(ea2, weT):
    e2, d = ea2.shape
    h = weT.shape[1]
    return pl.pallas_call(
        _edgeproj_body,
        grid=(e2 // _BLK,),
        in_specs=[
            pl.BlockSpec((_BLK, d), lambda i: (i, 0)),
            pl.BlockSpec((d, h), lambda i: (0, 0)),
        ],
        out_specs=pl.BlockSpec((_BLK, h), lambda i: (i, 0)),
        out_shape=jax.ShapeDtypeStruct((e2, h), jnp.bfloat16),
    )(ea2, weT)


def _logit_body(xls_ref, xrd_ref, e_ref, att_ref, out_ref):
    m = (xls_ref[...].astype(jnp.float32) + xrd_ref[...].astype(jnp.float32)
         + e_ref[...].astype(jnp.float32))
    g = jnp.where(m >= 0, m, _NEG_SLOPE * m)
    out_ref[...] = jnp.sum(g * att_ref[...], axis=1, keepdims=True)


def _edge_logits(xls, xrd, e2v, att):
    e2, h = xls.shape
    return pl.pallas_call(
        _logit_body,
        grid=(e2 // _BLK,),
        in_specs=[
            pl.BlockSpec((_BLK, h), lambda i: (i, 0)),
            pl.BlockSpec((_BLK, h), lambda i: (i, 0)),
            pl.BlockSpec((_BLK, h), lambda i: (i, 0)),
            pl.BlockSpec((1, h), lambda i: (0, 0)),
        ],
        out_specs=pl.BlockSpec((_BLK, 1), lambda i: (i, 0)),
        out_shape=jax.ShapeDtypeStruct((e2, 1), jnp.float32),
    )(xls, xrd, e2v, att.reshape(1, h))


def _alpha_body(z_ref, den_ref, xls_ref, out_ref):
    w = z_ref[...] / (den_ref[...] + 1e-16)
    out_ref[...] = (w * xls_ref[...].astype(jnp.float32)).astype(jnp.bfloat16)


def _edge_weighted(z, dend, xls):
    e2, h = xls.shape
    return pl.pallas_call(
        _alpha_body,
        grid=(e2 // _BLK,),
        in_specs=[
            pl.BlockSpec((_BLK, 1), lambda i: (i, 0)),
            pl.BlockSpec((_BLK, 1), lambda i: (i, 0)),
            pl.BlockSpec((_BLK, h), lambda i: (i, 0)),
        ],
        out_specs=pl.BlockSpec((_BLK, h), lambda i: (i, 0)),
        out_shape=jax.ShapeDtypeStruct((e2, h), jnp.bfloat16),
    )(z, dend, xls)


def _gatv2_layer(x, s2, d2, e2v, wl, bl, wr, br, att, b, n):
    xl, xr = _project(x, wl.T, bl, wr.T, br)
    xls = jnp.take(xl, s2, axis=0)
    xrd = jnp.take(xr, d2, axis=0)
    logit = _edge_logits(xls, xrd, e2v, att)[:, 0]
    mx = jax.ops.segment_max(logit, d2, num_segments=n)
    z = jnp.exp(logit - jnp.take(mx, d2))
    den = jax.ops.segment_sum(z, d2, num_segments=n)
    p = _edge_weighted(z[:, None], jnp.take(den, d2)[:, None], xls)
    out = jax.ops.segment_sum(p.astype(jnp.float32), d2, num_segments=n)
    return out + b


def kernel(x, edge_index, edge_attr,
           Wl1, bl1, Wr1, br1, We1, att1, b1,
           Wl2, bl2, Wr2, br2, We2, att2, b2,
           Wl3, bl3, Wr3, br3, We3, att3, b3):
    n = x.shape[0]
    src, dst = edge_index[0], edge_index[1]
    sums = jax.ops.segment_sum(edge_attr, dst, num_segments=n)
    cnt = jax.ops.segment_sum(jnp.ones((edge_attr.shape[0],), jnp.float32), dst,
                              num_segments=n)
    loop_ea = sums / jnp.maximum(cnt, 1.0)[:, None]
    loop_idx = jnp.arange(n, dtype=src.dtype)
    s2 = jnp.concatenate([src, loop_idx])
    d2 = jnp.concatenate([dst, loop_idx])
    ea2 = jnp.concatenate([edge_attr, loop_ea], axis=0)

    k1, k2 = jax.random.split(jax.random.key(42))

    h = _gatv2_layer(x, s2, d2, _edge_project(ea2, We1.T), Wl1, bl1, Wr1, br1,
                     att1, b1, n)
    h = jax.nn.relu(h)
    keep = jax.random.bernoulli(k1, 0.8, h.shape)
    h = jnp.where(keep, h / 0.8, jnp.zeros_like(h))

    h = _gatv2_layer(h, s2, d2, _edge_project(ea2, We2.T), Wl2, bl2, Wr2, br2,
                     att2, b2, n)
    h = jax.nn.relu(h)
    keep = jax.random.bernoulli(k2, 0.8, h.shape)
    h = jnp.where(keep, h / 0.8, jnp.zeros_like(h))

    h = _gatv2_layer(h, s2, d2, _edge_project(ea2, We3.T), Wl3, bl3, Wr3, br3,
                     att3, b3, n)
    return h


# speedup vs baseline: 1.4871x; 1.3979x over previous
import functools
import jax
import jax.numpy as jnp
from jax.experimental import pallas as pl

_NEG_SLOPE = 0.2
_BLK = 2000


def _proj_body(x_ref, wl_ref, bl_ref, wr_ref, br_ref, xl_ref, xr_ref):
    xb = x_ref[...]
    xl = jnp.dot(xb, wl_ref[...], preferred_element_type=jnp.float32) + bl_ref[...]
    xr = jnp.dot(xb, wr_ref[...], preferred_element_type=jnp.float32) + br_ref[...]
    xl_ref[...] = xl.astype(jnp.bfloat16)
    xr_ref[...] = xr.astype(jnp.bfloat16)


def _project(x, wlT, bl, wrT, br):
    n, f = x.shape
    h = wlT.shape[1]
    grid = (n // _BLK,)
    return pl.pallas_call(
        _proj_body,
        grid=grid,
        in_specs=[
            pl.BlockSpec((_BLK, f), lambda i: (i, 0)),
            pl.BlockSpec((f, h), lambda i: (0, 0)),
            pl.BlockSpec((1, h), lambda i: (0, 0)),
            pl.BlockSpec((f, h), lambda i: (0, 0)),
            pl.BlockSpec((1, h), lambda i: (0, 0)),
        ],
        out_specs=[
            pl.BlockSpec((_BLK, h), lambda i: (i, 0)),
            pl.BlockSpec((_BLK, h), lambda i: (i, 0)),
        ],
        out_shape=[
            jax.ShapeDtypeStruct((n, h), jnp.bfloat16),
            jax.ShapeDtypeStruct((n, h), jnp.bfloat16),
        ],
    )(x, wlT, bl.reshape(1, h), wrT, br.reshape(1, h))


def _edgeproj_body(ea_ref, we_ref, out_ref):
    out_ref[...] = jnp.dot(
        ea_ref[...], we_ref[...], preferred_element_type=jnp.float32
    ).astype(jnp.bfloat16)


def _edge_project(ea2, weT):
    e2, d = ea2.shape
    h = weT.shape[1]
    return pl.pallas_call(
        _edgeproj_body,
        grid=(e2 // _BLK,),
        in_specs=[
            pl.BlockSpec((_BLK, d), lambda i: (i, 0)),
            pl.BlockSpec((d, h), lambda i: (0, 0)),
        ],
        out_specs=pl.BlockSpec((_BLK, h), lambda i: (i, 0)),
        out_shape=jax.ShapeDtypeStruct((e2, h), jnp.bfloat16),
    )(ea2, weT)


def _logit_body(xls_ref, xrd_ref, e_ref, att_ref, out_ref):
    m = (xls_ref[...].astype(jnp.float32) + xrd_ref[...].astype(jnp.float32)
         + e_ref[...].astype(jnp.float32))
    g = jnp.where(m >= 0, m, _NEG_SLOPE * m)
    out_ref[...] = jnp.sum(g * att_ref[...], axis=1, keepdims=True)


def _edge_logits(xls, xrd, e2v, att):
    e2, h = xls.shape
    return pl.pallas_call(
        _logit_body,
        grid=(e2 // _BLK,),
        in_specs=[
            pl.BlockSpec((_BLK, h), lambda i: (i, 0)),
            pl.BlockSpec((_BLK, h), lambda i: (i, 0)),
            pl.BlockSpec((_BLK, h), lambda i: (i, 0)),
            pl.BlockSpec((1, h), lambda i: (0, 0)),
        ],
        out_specs=pl.BlockSpec((_BLK, 1), lambda i: (i, 0)),
        out_shape=jax.ShapeDtypeStruct((e2, 1), jnp.float32),
    )(xls, xrd, e2v, att.reshape(1, h))


def _gatv2_layer(x, s2, d2, e2v, wl, bl, wr, br, att, b, n):
    xl, xr = _project(x, wl.T, bl, wr.T, br)
    xls = jnp.take(xl, s2, axis=0)
    xrd = jnp.take(xr, d2, axis=0)
    logit = _edge_logits(xls, xrd, e2v, att)[:, 0]
    mx = jax.ops.segment_max(logit, d2, num_segments=n)
    z = jnp.exp(logit - jnp.take(mx, d2))
    den = jax.ops.segment_sum(z, d2, num_segments=n)
    num = jax.ops.segment_sum(z[:, None] * xls.astype(jnp.float32), d2,
                              num_segments=n)
    return num / (den + 1e-16)[:, None] + b


def kernel(x, edge_index, edge_attr,
           Wl1, bl1, Wr1, br1, We1, att1, b1,
           Wl2, bl2, Wr2, br2, We2, att2, b2,
           Wl3, bl3, Wr3, br3, We3, att3, b3):
    n = x.shape[0]
    src, dst = edge_index[0], edge_index[1]
    sums = jax.ops.segment_sum(edge_attr, dst, num_segments=n)
    cnt = jax.ops.segment_sum(jnp.ones((edge_attr.shape[0],), jnp.float32), dst,
                              num_segments=n)
    loop_ea = sums / jnp.maximum(cnt, 1.0)[:, None]
    loop_idx = jnp.arange(n, dtype=src.dtype)
    s2 = jnp.concatenate([src, loop_idx])
    d2 = jnp.concatenate([dst, loop_idx])
    ea2 = jnp.concatenate([edge_attr, loop_ea], axis=0)

    k1, k2 = jax.random.split(jax.random.key(42))

    h = _gatv2_layer(x, s2, d2, _edge_project(ea2, We1.T), Wl1, bl1, Wr1, br1,
                     att1, b1, n)
    h = jax.nn.relu(h)
    keep = jax.random.bernoulli(k1, 0.8, h.shape)
    h = jnp.where(keep, h / 0.8, jnp.zeros_like(h))

    h = _gatv2_layer(h, s2, d2, _edge_project(ea2, We2.T), Wl2, bl2, Wr2, br2,
                     att2, b2, n)
    h = jax.nn.relu(h)
    keep = jax.random.bernoulli(k2, 0.8, h.shape)
    h = jnp.where(keep, h / 0.8, jnp.zeros_like(h))

    h = _gatv2_layer(h, s2, d2, _edge_project(ea2, We3.T), Wl3, bl3, Wr3, br3,
                     att3, b3, n)
    return h
